# Initial kernel scaffold; baseline (speedup 1.0000x reference)
import jax
import jax.numpy as jnp
from jax.experimental import pallas as pl

_TOK = 1024
_K = 1024
_D = 64


def _vq_block(x_ref, w_ref, out_ref, idx_ref, loss_ref):
    xb = x_ref[0]
    w = w_ref[...]
    wsq = jnp.sum(w * w, axis=1, keepdims=True)
    mm = jax.lax.dot_general(
        w, xb, (((1,), (0,)), ((), ())),
        preferred_element_type=jnp.float32,
        precision=jax.lax.Precision.HIGHEST)
    d = wsq - 2.0 * mm
    dmin = jnp.min(d, axis=0, keepdims=True)
    iota = jax.lax.broadcasted_iota(jnp.int32, d.shape, 0)
    idx = jnp.min(jnp.where(d == dmin, iota, jnp.int32(2 ** 30)), axis=0)
    onehot = (iota == idx[None, :]).astype(jnp.float32)
    q = jax.lax.dot_general(
        w, onehot, (((0,), (0,)), ((), ())),
        preferred_element_type=jnp.float32,
        precision=jax.lax.Precision.HIGHEST)
    st = q - xb
    out_ref[0] = st + xb
    idx_ref[0, 0] = idx

    @pl.when((pl.program_id(0) == 0) & (pl.program_id(1) == 0))
    def _init():
        loss_ref[0, 0] = 0.0

    loss_ref[0, 0] += jnp.sum(st * st)


def kernel(x, weight):
    b, c, h, w, d = x.shape
    n_tok = h * w * d
    n_h = n_tok // _TOK
    xr = x.reshape(b, c, n_tok)
    out, idxf, loss = pl.pallas_call(
        _vq_block,
        grid=(b, n_h),
        in_specs=[
            pl.BlockSpec((1, _D, _TOK), lambda i, j: (i, 0, j)),
            pl.BlockSpec((_K, _D), lambda i, j: (0, 0)),
        ],
        out_specs=[
            pl.BlockSpec((1, _D, _TOK), lambda i, j: (i, 0, j)),
            pl.BlockSpec((1, 1, _TOK), lambda i, j: (i * n_h + j, 0, 0)),
            pl.BlockSpec((1, 1), lambda i, j: (0, 0)),
        ],
        out_shape=[
            jax.ShapeDtypeStruct((b, c, n_tok), jnp.float32),
            jax.ShapeDtypeStruct((b * n_h, 1, _TOK), jnp.int32),
            jax.ShapeDtypeStruct((1, 1), jnp.float32),
        ],
    )(xr, weight)
    quantized_st = out.reshape(b, c, h, w, d)
    embed_idx = idxf.reshape(b, h, w, d)
    latent_loss = 0.25 * (loss[0, 0] / (b * c * h * w * d))
    return quantized_st, embed_idx, latent_loss


# speedup vs baseline: 1.2773x; 1.2773x over previous
import jax
import jax.numpy as jnp
from jax.experimental import pallas as pl

_TOK = 1024
_K = 1024
_D = 64


def _vq_block(x_ref, w_ref, out_ref, idx_ref, loss_ref):
    xb = x_ref[0]
    w = w_ref[...]
    wsq = jnp.sum(w * w, axis=1, keepdims=True)
    fsq = jnp.sum(xb * xb, axis=0, keepdims=True)
    mm = jax.lax.dot_general(
        w, xb, (((1,), (0,)), ((), ())),
        preferred_element_type=jnp.float32,
        precision=jax.lax.Precision.DEFAULT)
    d = (fsq - 2.0 * mm) + wsq
    dmin = jnp.min(d, axis=0, keepdims=True)
    iota = jax.lax.broadcasted_iota(jnp.int32, d.shape, 0)
    idx = jnp.min(jnp.where(d == dmin, iota, jnp.int32(2 ** 30)), axis=0)
    onehot = (iota == idx[None, :]).astype(jnp.float32)
    q = jax.lax.dot_general(
        w, onehot, (((0,), (0,)), ((), ())),
        preferred_element_type=jnp.float32,
        precision=jax.lax.Precision.HIGHEST)
    st = q - xb
    out_ref[0] = st + xb
    idx_ref[0, 0] = idx

    @pl.when((pl.program_id(0) == 0) & (pl.program_id(1) == 0))
    def _init():
        loss_ref[...] = jnp.zeros_like(loss_ref)

    loss_ref[...] += jnp.sum(st * st, keepdims=True)


def kernel(x, weight):
    b, c, h, w, d = x.shape
    n_tok = h * w * d
    n_h = n_tok // _TOK
    xr = x.reshape(b, c, n_tok)
    out, idxf, loss = pl.pallas_call(
        _vq_block,
        grid=(b, n_h),
        in_specs=[
            pl.BlockSpec((1, _D, _TOK), lambda i, j: (i, 0, j)),
            pl.BlockSpec((_K, _D), lambda i, j: (0, 0)),
        ],
        out_specs=[
            pl.BlockSpec((1, _D, _TOK), lambda i, j: (i, 0, j)),
            pl.BlockSpec((1, 1, _TOK), lambda i, j: (i * n_h + j, 0, 0)),
            pl.BlockSpec((1, 1), lambda i, j: (0, 0)),
        ],
        out_shape=[
            jax.ShapeDtypeStruct((b, c, n_tok), jnp.float32),
            jax.ShapeDtypeStruct((b * n_h, 1, _TOK), jnp.int32),
            jax.ShapeDtypeStruct((1, 1), jnp.float32),
        ],
    )(xr, weight)
    quantized_st = out.reshape(b, c, h, w, d)
    embed_idx = idxf.reshape(b, h, w, d)
    latent_loss = 0.25 * (loss[0, 0] / (b * c * h * w * d))
    return quantized_st, embed_idx, latent_loss


# speedup vs baseline: 1.4813x; 1.1597x over previous
import jax
import jax.numpy as jnp
from jax.experimental import pallas as pl

_TOK = 1024
_K = 1024
_D = 64


def _vq_block(x_ref, w_ref, probe_ref, out_ref, idx_ref, loss_ref):
    xb = x_ref[0]
    w = w_ref[...]
    wsq = jnp.sum(w * w, axis=1, keepdims=True)
    fsq = jnp.sum(xb * xb, axis=0, keepdims=True)
    mm = jax.lax.dot_general(
        w, xb, (((1,), (0,)), ((), ())),
        preferred_element_type=jnp.float32,
        precision=jax.lax.Precision.DEFAULT)
    d = (fsq - 2.0 * mm) + wsq
    dmin = jnp.min(d, axis=0, keepdims=True)
    oh = jnp.where(d == dmin, 1.0, 0.0).astype(jnp.bfloat16)

    tri = jax.lax.dot_general(
        probe_ref[...], oh, (((1,), (0,)), ((), ())),
        preferred_element_type=jnp.float32)
    has_tie = jnp.max(tri[0:1, :]) > 1.5

    def _tie_path(_):
        iota = jax.lax.broadcasted_iota(jnp.int32, d.shape, 0)
        idx_ex = jnp.min(jnp.where(d == dmin, iota, jnp.int32(2 ** 30)), axis=0)
        oh_ex = jnp.where(iota == idx_ex[None, :], 1.0, 0.0)
        return idx_ex, oh_ex.astype(jnp.bfloat16)

    def _fast_path(_):
        return (4.0 * tri[1, :] + tri[2, :]).astype(jnp.int32), oh

    idx, ohf = jax.lax.cond(has_tie, _tie_path, _fast_path, None)

    w_hi = w.astype(jnp.bfloat16)
    r1 = w - w_hi.astype(jnp.float32)
    w_mid = r1.astype(jnp.bfloat16)
    w_lo = (r1 - w_mid.astype(jnp.float32)).astype(jnp.bfloat16)

    def _gmm(wp):
        return jax.lax.dot_general(
            wp, ohf, (((0,), (0,)), ((), ())),
            preferred_element_type=jnp.float32)

    q = (_gmm(w_hi) + _gmm(w_mid)) + _gmm(w_lo)
    st = q - xb
    out_ref[0] = st + xb
    idx_ref[0, 0] = idx

    @pl.when((pl.program_id(0) == 0) & (pl.program_id(1) == 0))
    def _init():
        loss_ref[...] = jnp.zeros_like(loss_ref)

    loss_ref[...] += jnp.sum(st * st, keepdims=True)


def kernel(x, weight):
    b, c, h, w, d = x.shape
    n_tok = h * w * d
    n_h = n_tok // _TOK
    xr = x.reshape(b, c, n_tok)
    kvec = jnp.arange(_K, dtype=jnp.int32)[None, :]
    probe = jnp.concatenate(
        [jnp.ones((1, _K), jnp.float32),
         (kvec // 4).astype(jnp.float32),
         (kvec % 4).astype(jnp.float32)], axis=0).astype(jnp.bfloat16)
    out, idxf, loss = pl.pallas_call(
        _vq_block,
        grid=(b, n_h),
        in_specs=[
            pl.BlockSpec((1, _D, _TOK), lambda i, j: (i, 0, j)),
            pl.BlockSpec((_K, _D), lambda i, j: (0, 0)),
            pl.BlockSpec((3, _K), lambda i, j: (0, 0)),
        ],
        out_specs=[
            pl.BlockSpec((1, _D, _TOK), lambda i, j: (i, 0, j)),
            pl.BlockSpec((1, 1, _TOK), lambda i, j: (i * n_h + j, 0, 0)),
            pl.BlockSpec((1, 1), lambda i, j: (0, 0)),
        ],
        out_shape=[
            jax.ShapeDtypeStruct((b, c, n_tok), jnp.float32),
            jax.ShapeDtypeStruct((b * n_h, 1, _TOK), jnp.int32),
            jax.ShapeDtypeStruct((1, 1), jnp.float32),
        ],
    )(xr, weight, probe)
    quantized_st = out.reshape(b, c, h, w, d)
    embed_idx = idxf.reshape(b, h, w, d)
    latent_loss = 0.25 * (loss[0, 0] / (b * c * h * w * d))
    return quantized_st, embed_idx, latent_loss


# speedup vs baseline: 1.6230x; 1.0956x over previous
import jax
import jax.numpy as jnp
from jax.experimental import pallas as pl

_TOK = 1024
_CHUNK = 512
_K = 1024
_D = 64


def _vq_block(x_ref, w_ref, out_ref, idx_ref, loss_ref):
    w = w_ref[...]
    wsq = jnp.sum(w * w, axis=1, keepdims=True)

    w_hi = w.astype(jnp.bfloat16)
    r1 = w - w_hi.astype(jnp.float32)
    w_mid = r1.astype(jnp.bfloat16)
    w_lo = (r1 - w_mid.astype(jnp.float32)).astype(jnp.bfloat16)

    loss_parts = []
    for c in range(_TOK // _CHUNK):
        sl = pl.ds(c * _CHUNK, _CHUNK)
        xb = x_ref[0, :, sl]
        fsq = jnp.sum(xb * xb, axis=0, keepdims=True)
        mm = jax.lax.dot_general(
            w, xb, (((1,), (0,)), ((), ())),
            preferred_element_type=jnp.float32,
            precision=jax.lax.Precision.DEFAULT)
        d = (fsq - 2.0 * mm) + wsq
        dmin = jnp.min(d, axis=0, keepdims=True)
        iota = jax.lax.broadcasted_iota(jnp.int32, d.shape, 0)
        idx = jnp.min(jnp.where(d == dmin, iota, jnp.int32(2 ** 30)), axis=0)
        oh = jnp.where(iota == idx[None, :], 1.0, 0.0).astype(jnp.bfloat16)

        def _gmm(wp, oh=oh):
            return jax.lax.dot_general(
                wp, oh, (((0,), (0,)), ((), ())),
                preferred_element_type=jnp.float32)

        q = (_gmm(w_hi) + _gmm(w_mid)) + _gmm(w_lo)
        st = q - xb
        out_ref[0, :, sl] = st + xb
        idx_ref[0, 0, sl] = idx
        loss_parts.append(jnp.sum(st * st, keepdims=True))

    @pl.when((pl.program_id(0) == 0) & (pl.program_id(1) == 0))
    def _init():
        loss_ref[...] = jnp.zeros_like(loss_ref)

    total = loss_parts[0]
    for p in loss_parts[1:]:
        total = total + p
    loss_ref[...] += total


def kernel(x, weight):
    b, c, h, w, d = x.shape
    n_tok = h * w * d
    n_h = n_tok // _TOK
    xr = x.reshape(b, c, n_tok)
    out, idxf, loss = pl.pallas_call(
        _vq_block,
        grid=(b, n_h),
        in_specs=[
            pl.BlockSpec((1, _D, _TOK), lambda i, j: (i, 0, j)),
            pl.BlockSpec((_K, _D), lambda i, j: (0, 0)),
        ],
        out_specs=[
            pl.BlockSpec((1, _D, _TOK), lambda i, j: (i, 0, j)),
            pl.BlockSpec((1, 1, _TOK), lambda i, j: (i * n_h + j, 0, 0)),
            pl.BlockSpec((1, 1), lambda i, j: (0, 0)),
        ],
        out_shape=[
            jax.ShapeDtypeStruct((b, c, n_tok), jnp.float32),
            jax.ShapeDtypeStruct((b * n_h, 1, _TOK), jnp.int32),
            jax.ShapeDtypeStruct((1, 1), jnp.float32),
        ],
    )(xr, weight)
    quantized_st = out.reshape(b, c, h, w, d)
    embed_idx = idxf.reshape(b, h, w, d)
    latent_loss = 0.25 * (loss[0, 0] / (b * c * h * w * d))
    return quantized_st, embed_idx, latent_loss


# speedup vs baseline: 1.7301x; 1.0660x over previous
import jax
import jax.numpy as jnp
from jax.experimental import pallas as pl

_TOK = 2048
_CHUNK = 512
_K = 1024
_D = 64


def _vq_block(x_ref, w_ref, out_ref, idx_ref, loss_ref):
    w = w_ref[...]
    wsq = jnp.sum(w * w, axis=1, keepdims=True)

    w_hi = w.astype(jnp.bfloat16)
    r1 = w - w_hi.astype(jnp.float32)
    w_mid = r1.astype(jnp.bfloat16)
    w_lo = (r1 - w_mid.astype(jnp.float32)).astype(jnp.bfloat16)

    loss_parts = []
    for c in range(_TOK // _CHUNK):
        sl = pl.ds(c * _CHUNK, _CHUNK)
        xb = x_ref[0, :, sl]
        fsq = jnp.sum(xb * xb, axis=0, keepdims=True)
        mm = jax.lax.dot_general(
            w, xb, (((1,), (0,)), ((), ())),
            preferred_element_type=jnp.float32,
            precision=jax.lax.Precision.DEFAULT)
        d = (fsq - 2.0 * mm) + wsq
        dmin = jnp.min(d, axis=0, keepdims=True)
        iota = jax.lax.broadcasted_iota(jnp.int32, d.shape, 0)
        idx = jnp.min(jnp.where(d == dmin, iota, jnp.int32(2 ** 30)), axis=0)
        oh = jnp.where(iota == idx[None, :], 1.0, 0.0).astype(jnp.bfloat16)

        def _gmm(wp, oh=oh):
            return jax.lax.dot_general(
                wp, oh, (((0,), (0,)), ((), ())),
                preferred_element_type=jnp.float32)

        q = (_gmm(w_hi) + _gmm(w_mid)) + _gmm(w_lo)
        st = q - xb
        out_ref[0, :, sl] = st + xb
        idx_ref[0, 0, sl] = idx
        loss_parts.append(jnp.sum(st * st, keepdims=True))

    @pl.when((pl.program_id(0) == 0) & (pl.program_id(1) == 0))
    def _init():
        loss_ref[...] = jnp.zeros_like(loss_ref)

    total = loss_parts[0]
    for p in loss_parts[1:]:
        total = total + p
    loss_ref[...] += total


def kernel(x, weight):
    b, c, h, w, d = x.shape
    n_tok = h * w * d
    n_h = n_tok // _TOK
    xr = x.reshape(b, c, n_tok)
    out, idxf, loss = pl.pallas_call(
        _vq_block,
        grid=(b, n_h),
        in_specs=[
            pl.BlockSpec((1, _D, _TOK), lambda i, j: (i, 0, j)),
            pl.BlockSpec((_K, _D), lambda i, j: (0, 0)),
        ],
        out_specs=[
            pl.BlockSpec((1, _D, _TOK), lambda i, j: (i, 0, j)),
            pl.BlockSpec((1, 1, _TOK), lambda i, j: (i * n_h + j, 0, 0)),
            pl.BlockSpec((1, 1), lambda i, j: (0, 0)),
        ],
        out_shape=[
            jax.ShapeDtypeStruct((b, c, n_tok), jnp.float32),
            jax.ShapeDtypeStruct((b * n_h, 1, _TOK), jnp.int32),
            jax.ShapeDtypeStruct((1, 1), jnp.float32),
        ],
    )(xr, weight)
    quantized_st = out.reshape(b, c, h, w, d)
    embed_idx = idxf.reshape(b, h, w, d)
    latent_loss = 0.25 * (loss[0, 0] / (b * c * h * w * d))
    return quantized_st, embed_idx, latent_loss


# speedup vs baseline: 2.3910x; 1.3820x over previous
import jax
import jax.numpy as jnp
from jax.experimental import pallas as pl

_TOK = 2048
_CHUNK = 512
_K = 1024
_D = 64


def _vq_block(x_ref, w_ref, out_ref, idx_ref, loss_ref):
    w = w_ref[...]
    wsq = jnp.sum(w * w, axis=1, keepdims=True)
    w2 = w + w
    iota = jax.lax.broadcasted_iota(jnp.int32, (_K, _CHUNK), 0)

    w_hi = w.astype(jnp.bfloat16)
    r1 = w - w_hi.astype(jnp.float32)
    w_mid = r1.astype(jnp.bfloat16)
    w_lo = (r1 - w_mid.astype(jnp.float32)).astype(jnp.bfloat16)
    w3 = jnp.concatenate([w_hi, w_mid, w_lo], axis=1)

    loss_parts = []
    for c in range(_TOK // _CHUNK):
        sl = pl.ds(c * _CHUNK, _CHUNK)
        xb = x_ref[0, :, sl]
        fsq = jnp.sum(xb * xb, axis=0, keepdims=True)
        mm2 = jax.lax.dot_general(
            w2, xb, (((1,), (0,)), ((), ())),
            preferred_element_type=jnp.float32,
            precision=jax.lax.Precision.DEFAULT)
        d = (fsq - mm2) + wsq
        dmin = jnp.min(d, axis=0, keepdims=True)
        idx = jnp.min(jnp.where(d == dmin, iota, jnp.int32(2 ** 30)), axis=0)
        oh = jnp.where(iota == idx[None, :], 1.0, 0.0).astype(jnp.bfloat16)

        g3 = jax.lax.dot_general(
            w3, oh, (((0,), (0,)), ((), ())),
            preferred_element_type=jnp.float32)
        q = (g3[0:_D] + g3[_D:2 * _D]) + g3[2 * _D:3 * _D]
        st = q - xb
        out_ref[0, :, sl] = st + xb
        idx_ref[0, 0, sl] = idx
        loss_parts.append(jnp.sum(st * st, keepdims=True))

    @pl.when((pl.program_id(0) == 0) & (pl.program_id(1) == 0))
    def _init():
        loss_ref[...] = jnp.zeros_like(loss_ref)

    total = loss_parts[0]
    for p in loss_parts[1:]:
        total = total + p
    loss_ref[...] += total


def kernel(x, weight):
    b, c, h, w, d = x.shape
    n_tok = h * w * d
    n_h = n_tok // _TOK
    xr = x.reshape(b, c, n_tok)
    out, idxf, loss = pl.pallas_call(
        _vq_block,
        grid=(b, n_h),
        in_specs=[
            pl.BlockSpec((1, _D, _TOK), lambda i, j: (i, 0, j)),
            pl.BlockSpec((_K, _D), lambda i, j: (0, 0)),
        ],
        out_specs=[
            pl.BlockSpec((1, _D, _TOK), lambda i, j: (i, 0, j)),
            pl.BlockSpec((1, 1, _TOK), lambda i, j: (i * n_h + j, 0, 0)),
            pl.BlockSpec((1, 1), lambda i, j: (0, 0)),
        ],
        out_shape=[
            jax.ShapeDtypeStruct((b, c, n_tok), jnp.float32),
            jax.ShapeDtypeStruct((b * n_h, 1, _TOK), jnp.int32),
            jax.ShapeDtypeStruct((1, 1), jnp.float32),
        ],
    )(xr, weight)
    quantized_st = out.reshape(b, c, h, w, d)
    embed_idx = idxf.reshape(b, h, w, d)
    latent_loss = 0.25 * (loss[0, 0] / (b * c * h * w * d))
    return quantized_st, embed_idx, latent_loss
